# baseline (speedup 1.0000x reference)
import functools

import numpy as np
import jax
import jax.numpy as jnp
from jax import lax
from jax.experimental import pallas as pl
from jax.experimental.pallas import tpu as pltpu
from jax.experimental.pallas import tpu_sc as plsc

B = 1024
D = 32
BI = 64

_K_TILE = np.tile(np.eye(D, dtype=np.float32), (1, B))


def _sc_gather(w_r, idx):
    info = plsc.get_sparse_core_info()
    nw = info.num_cores * info.num_subcores
    bpw = B // nw

    mesh = plsc.VectorSubcoreMesh(core_axis_name="c", subcore_axis_name="s")

    @functools.partial(
        pl.kernel,
        out_type=jax.ShapeDtypeStruct((B, D), jnp.float32),
        mesh=mesh,
        scratch_types=[
            pltpu.VMEM((bpw,), jnp.int32),
            pltpu.VMEM((bpw, D), jnp.float32),
            pltpu.SemaphoreType.DMA,
        ],
        compiler_params=pltpu.CompilerParams(use_tc_tiling_on_sc=False),
    )
    def k(table_hbm, idx_hbm, out_hbm, idx_v, rows_v, sem):
        wid = lax.axis_index("s") * info.num_cores + lax.axis_index("c")
        base = wid * bpw
        pltpu.sync_copy(idx_hbm.at[pl.ds(base, bpw)], idx_v)
        pltpu.async_copy(table_hbm.at[idx_v], rows_v, sem).wait()
        pltpu.sync_copy(rows_v, out_hbm.at[pl.ds(base, bpw)])

    return k(w_r, idx)


def _proj_body(g_ref, e_ref, e2_ref, k_ref, out_ref):
    g = g_ref[...]
    inv = lax.rsqrt(jnp.sum(g * g, axis=1, keepdims=True))
    n = g * inv
    dots_rep = jnp.dot(n, e2_ref[...], preferred_element_type=jnp.float32)
    n_tile = jnp.dot(n, k_ref[...], preferred_element_type=jnp.float32)
    out_ref[...] = e_ref[...] - dots_rep * n_tile


def _project(gathered, e_flat, e2, k_const):
    return pl.pallas_call(
        _proj_body,
        grid=(B // BI,),
        in_specs=[
            pl.BlockSpec((BI, D), lambda i: (i, 0)),
            pl.BlockSpec((1, B * D), lambda i: (0, 0)),
            pl.BlockSpec((D, B * D), lambda i: (0, 0)),
            pl.BlockSpec((D, B * D), lambda i: (0, 0)),
        ],
        out_specs=pl.BlockSpec((BI, B * D), lambda i: (i, 0)),
        out_shape=jax.ShapeDtypeStruct((B, B * D), jnp.float32),
        compiler_params=pltpu.CompilerParams(
            dimension_semantics=("arbitrary",),
        ),
    )(gathered, e_flat, e2, k_const)


def kernel(batch_e_emb, batch_r_id, w_r):
    idx = batch_r_id.reshape(B).astype(jnp.int32)
    gathered = _sc_gather(w_r, idx)
    e_flat = batch_e_emb.reshape(1, B * D)
    e2 = jnp.repeat(batch_e_emb.T, D, axis=1)
    out2d = _project(gathered, e_flat, e2, jnp.asarray(_K_TILE))
    return out2d.reshape(B, B, D)


# speedup vs baseline: 2.7422x; 2.7422x over previous
import jax
import jax.numpy as jnp
from jax import lax
from jax.experimental import pallas as pl
from jax.experimental.pallas import tpu as pltpu

B = 1024
D = 32
BI = 8


def _body(idx_ref, *refs):
    col_refs = refs[:BI]
    eT_ref, out_ref = refs[BI], refs[BI + 1]
    i = pl.program_id(0)

    cols = []
    for t in range(BI):
        lane = lax.rem(idx_ref[i * BI + t], 128)
        onehot = lax.broadcasted_iota(jnp.int32, (D, 128), 1) == lane
        x = col_refs[t][...]
        cols.append(jnp.sum(jnp.where(onehot, x, 0.0), axis=1, keepdims=True))
    gT = jnp.concatenate(cols, axis=1)
    eye = (lax.broadcasted_iota(jnp.int32, (D, D), 0)
           == lax.broadcasted_iota(jnp.int32, (D, D), 1)).astype(jnp.float32)
    g = lax.dot_general(gT, eye, (((0,), (0,)), ((), ())),
                        preferred_element_type=jnp.float32)
    n = g * lax.rsqrt(jnp.sum(g * g, axis=1, keepdims=True))
    eT = eT_ref[...]
    dots = jnp.dot(n, eT, preferred_element_type=jnp.float32)
    for d in range(D):
        out_ref[:, d, :] = eT[d : d + 1, :] - n[:, d : d + 1] * dots


def kernel(batch_e_emb, batch_r_id, w_r):
    idx = batch_r_id.reshape(B).astype(jnp.int32)
    eT = batch_e_emb.T
    wT = w_r.T

    def col_spec(t):
        return pl.BlockSpec(
            (D, 128), lambda i, idx_ref, t=t: (0, idx_ref[i * BI + t] // 128)
        )

    grid_spec = pltpu.PrefetchScalarGridSpec(
        num_scalar_prefetch=1,
        grid=(B // BI,),
        in_specs=[col_spec(t) for t in range(BI)]
        + [pl.BlockSpec((D, B), lambda i, idx_ref: (0, 0))],
        out_specs=pl.BlockSpec((BI, D, B), lambda i, idx_ref: (i, 0, 0)),
        scratch_shapes=[],
    )
    out_idj = pl.pallas_call(
        _body,
        grid_spec=grid_spec,
        out_shape=jax.ShapeDtypeStruct((B, D, B), jnp.float32),
        compiler_params=pltpu.CompilerParams(
            dimension_semantics=("arbitrary",),
        ),
    )(idx, *([wT] * BI), eT)
    return out_idj.transpose(0, 2, 1)
